# baseline (speedup 1.0000x reference)
import functools

import jax
import jax.numpy as jnp
from jax import lax
from jax.experimental import pallas as pl
from jax.experimental.pallas import tpu as pltpu
from jax.experimental.pallas import tpu_sc as plsc

_N_BINS = 15
_L = 16
_ROWS, _COLS = 16384, 2048
_NW = 32
_STRIPE = _ROWS // _NW

_mesh = plsc.VectorSubcoreMesh(core_axis_name="c", subcore_axis_name="s")


@functools.partial(
    pl.kernel,
    mesh=_mesh,
    out_type=(
        jax.ShapeDtypeStruct((_ROWS, _COLS), jnp.float32),
        jax.ShapeDtypeStruct((1,), jnp.float32),
        jax.ShapeDtypeStruct((1,), jnp.float32),
        jax.ShapeDtypeStruct((_N_BINS,), jnp.float32),
    ),
    scratch_types=[
        pltpu.VMEM((_L,), jnp.float32),
        pltpu.VMEM((_L,), jnp.float32),
        pltpu.VMEM((_L,), jnp.float32),
        pltpu.VMEM((_L,), jnp.float32),
        pltpu.VMEM((_L,), jnp.float32),
        pltpu.VMEM((_L,), jnp.float32),
        pltpu.SemaphoreType.DMA,
        pltpu.SemaphoreType.DMA,
    ],
    compiler_params=pltpu.CompilerParams(needs_layout_passes=False),
)
def _sc_forward(temp_hbm, bc_hbm, bt_hbm, x_hbm,
                xout_hbm, ece_hbm, tout_hbm, acc_hbm,
                bc_v, bt_v, t_v, acc_v, ece_v, tout_v, sem, xsem):
    wid = lax.axis_index("s") * 2 + lax.axis_index("c")
    base = wid * _STRIPE
    cp_x = pltpu.make_async_copy(x_hbm.at[pl.ds(base, _STRIPE)],
                                 xout_hbm.at[pl.ds(base, _STRIPE)], xsem)
    cp_x.start()

    @pl.when(wid == 0)
    def _():
        zero = jnp.zeros((_L,), jnp.float32)
        bc_v[...] = zero
        bt_v[...] = zero
        cp_bc = pltpu.make_async_copy(bc_hbm, bc_v.at[pl.ds(0, _N_BINS)], sem)
        cp_bt = pltpu.make_async_copy(bt_hbm, bt_v.at[pl.ds(0, _N_BINS)], sem)
        cp_t = pltpu.make_async_copy(temp_hbm, t_v.at[pl.ds(0, 1)], sem)
        cp_bc.start()
        cp_bt.start()
        cp_t.start()
        cp_bc.wait()
        cp_bt.wait()
        cp_t.wait()

        bc = bc_v[...]
        bt = bt_v[...]
        acc = bc / (bt + 1e-8)
        lane = lax.iota(jnp.int32, _L)
        mask = lane < _N_BINS
        conf = lane.astype(jnp.float32) * (1.0 / (_N_BINS - 1)) + (0.5 / _N_BINS)
        btm = jnp.where(mask, bt, 0.0)
        n = jnp.sum(btm)
        s = jnp.sum(jnp.where(mask, bt * jnp.abs(acc - conf), 0.0))
        nv = jnp.full((_L,), n)
        sv = jnp.full((_L,), s)
        ece = jnp.where(nv > 0.0, sv / jnp.maximum(nv, 1e-8), 0.0)
        acc_v[...] = acc
        ece_v[...] = ece
        tout_v[...] = jnp.clip(t_v[...], 0.1, 10.0)

        cp_acc = pltpu.make_async_copy(acc_v.at[pl.ds(0, _N_BINS)], acc_hbm, sem)
        cp_ece = pltpu.make_async_copy(ece_v.at[pl.ds(0, 1)], ece_hbm, sem)
        cp_to = pltpu.make_async_copy(tout_v.at[pl.ds(0, 1)], tout_hbm, sem)
        cp_acc.start()
        cp_ece.start()
        cp_to.start()
        cp_acc.wait()
        cp_ece.wait()
        cp_to.wait()

    cp_x.wait()


def kernel(x, temperature, platt_a, platt_b, bin_correct, bin_total):
    xout, ece, temp, acc = _sc_forward(
        temperature.reshape(1), bin_correct, bin_total, x)
    return (xout, ece.reshape(()), temp.reshape(()), acc)


# speedup vs baseline: 47.6382x; 47.6382x over previous
import jax
import jax.numpy as jnp
from jax.experimental import pallas as pl
from jax.experimental.pallas import tpu as pltpu

_N_BINS = 15
_ROWS, _COLS = 16384, 2048
_BLK = 512


def _fused_kernel(temp_ref, bc_ref, bt_ref, x_ref,
                  xout_ref, ece_ref, tout_ref, acc_ref):
    xout_ref[...] = x_ref[...]

    @pl.when(pl.program_id(0) == 0)
    def _stats():
        n = jnp.float32(0.0)
        for i in range(_N_BINS):
            n = n + bt_ref[i]
        s = jnp.float32(0.0)
        for i in range(_N_BINS):
            bc = bc_ref[i]
            bt = bt_ref[i]
            acc = bc / (bt + 1e-8)
            acc_ref[i] = acc
            conf = i / (_N_BINS - 1.0) + 0.5 / _N_BINS
            s = s + bt * jnp.abs(acc - conf)
        ece_ref[0] = jnp.where(n > 0.0, s / jnp.maximum(n, 1e-8), 0.0)
        tout_ref[0] = jnp.clip(temp_ref[0], 0.1, 10.0)


def kernel(x, temperature, platt_a, platt_b, bin_correct, bin_total):
    xout, ece, temp, acc = pl.pallas_call(
        _fused_kernel,
        grid=(_ROWS // _BLK,),
        out_shape=(
            jax.ShapeDtypeStruct((_ROWS, _COLS), jnp.float32),
            jax.ShapeDtypeStruct((1,), jnp.float32),
            jax.ShapeDtypeStruct((1,), jnp.float32),
            jax.ShapeDtypeStruct((_N_BINS,), jnp.float32),
        ),
        in_specs=[
            pl.BlockSpec(memory_space=pltpu.SMEM),
            pl.BlockSpec(memory_space=pltpu.SMEM),
            pl.BlockSpec(memory_space=pltpu.SMEM),
            pl.BlockSpec((_BLK, _COLS), lambda i: (i, 0)),
        ],
        out_specs=(
            pl.BlockSpec((_BLK, _COLS), lambda i: (i, 0)),
            pl.BlockSpec(memory_space=pltpu.SMEM),
            pl.BlockSpec(memory_space=pltpu.SMEM),
            pl.BlockSpec(memory_space=pltpu.SMEM),
        ),
    )(temperature.reshape(1), bin_correct, bin_total, x)
    return (xout, ece.reshape(()), temp.reshape(()), acc)
